# baseline (speedup 1.0000x reference)
import jax
import jax.numpy as jnp
from jax import lax
from jax.experimental import pallas as pl
from jax.experimental.pallas import tpu as pltpu
from jax.experimental.pallas import tpu_sc as plsc

D_MODEL = 768
N_TOKENS = 4 * 4096
NC, NS = 2, 16
NW = NC * NS

N_TC = 5120
N_SC = N_TOKENS - N_TC
BPW = N_SC // NW
CHUNK = 32
NCH = BPW // CHUNK
NBUF = 4
ROWS_PER_STEP = 256
TC_STEPS = N_TC // ROWS_PER_STEP


def _make_sc_gather():
    mesh = plsc.VectorSubcoreMesh(core_axis_name="c", subcore_axis_name="s")

    def body(tokens_hbm, table_hbm, out_hbm, idx_v, bufs, gsems, osems):
        wid = lax.axis_index("s") * NC + lax.axis_index("c")
        base = wid * BPW
        pltpu.sync_copy(tokens_hbm.at[pl.ds(base, BPW)], idx_v)

        def start_gather(i):
            b = i % NBUF
            return pltpu.async_copy(
                table_hbm.at[idx_v.at[pl.ds(i * CHUNK, CHUNK)]],
                bufs[b], gsems[b])

        def start_out(i):
            b = i % NBUF
            return pltpu.async_copy(
                bufs[b], out_hbm.at[pl.ds(base + i * CHUNK, CHUNK)],
                osems[b])

        gh = {i: start_gather(i) for i in range(min(NBUF, NCH))}
        oh = {}
        for i in range(NCH):
            gh[i].wait()
            oh[i] = start_out(i)
            if i + NBUF < NCH:
                oh[i].wait()
                gh[i + NBUF] = start_gather(i + NBUF)
        for i in range(max(0, NCH - NBUF), NCH):
            oh[i].wait()

    return pl.kernel(
        body,
        out_type=jax.ShapeDtypeStruct((N_TOKENS, D_MODEL), jnp.float32),
        mesh=mesh,
        scratch_types=[
            pltpu.VMEM((BPW,), jnp.int32),
            tuple(pltpu.VMEM((CHUNK, D_MODEL), jnp.float32)
                  for _ in range(NBUF)),
            tuple(pltpu.SemaphoreType.DMA for _ in range(NBUF)),
            tuple(pltpu.SemaphoreType.DMA for _ in range(NBUF)),
        ],
    )


def _make_tc_gather():
    def body(tok_smem, table_any, out_vmem, sem):
        g = pl.program_id(0)
        handles = []
        for r in range(ROWS_PER_STEP):
            idx = tok_smem[g * ROWS_PER_STEP + r]
            handles.append(
                pltpu.make_async_copy(
                    table_any.at[pl.ds(idx, 1), :],
                    out_vmem.at[pl.ds(r, 1), :],
                    sem,
                )
            )
            handles[-1].start()
        for h in handles:
            h.wait()

    grid_spec = pltpu.PrefetchScalarGridSpec(
        num_scalar_prefetch=1,
        grid=(TC_STEPS,),
        in_specs=[pl.BlockSpec(memory_space=pl.ANY)],
        out_specs=pl.BlockSpec((ROWS_PER_STEP, D_MODEL), lambda i, tok: (i, 0)),
        scratch_shapes=[pltpu.SemaphoreType.DMA],
    )

    def run(tokens_tc, W_E):
        return pl.pallas_call(
            body,
            grid_spec=grid_spec,
            out_shape=jax.ShapeDtypeStruct((N_TC, D_MODEL), jnp.float32),
        )(tokens_tc, W_E)

    return run


_sc_gather = _make_sc_gather()
_tc_gather = _make_tc_gather()


@jax.jit
def _gather_all(tokens_flat, W_E):
    emb_full = _sc_gather(tokens_flat, W_E)
    emb_tc = _tc_gather(tokens_flat[N_SC:], W_E)
    return lax.dynamic_update_slice(emb_full, emb_tc, (N_SC, 0))


def kernel(tokens, W_E):
    B, S = tokens.shape
    tokens_flat = tokens.reshape(-1).astype(jnp.int32)
    emb = _gather_all(tokens_flat, W_E)
    return (tokens, emb.reshape(B, S, D_MODEL))


# speedup vs baseline: 1.0044x; 1.0044x over previous
import jax
import jax.numpy as jnp
from jax import lax
from jax.experimental import pallas as pl
from jax.experimental.pallas import tpu as pltpu
from jax.experimental.pallas import tpu_sc as plsc

D_MODEL = 768
N_TOKENS = 4 * 4096
NC, NS = 2, 16
NW = NC * NS

N_TC = 5120
N_SC = N_TOKENS - N_TC
BPW = N_SC // NW
CHUNK = 32
NCH = BPW // CHUNK
NBUF = 4
ROWS_PER_STEP = 256
TC_STEPS = N_TC // ROWS_PER_STEP


def _make_sc_gather():
    mesh = plsc.VectorSubcoreMesh(core_axis_name="c", subcore_axis_name="s")

    def body(tokens_hbm, table_hbm, out_hbm, idx_v, bufs, gsems, osems):
        wid = lax.axis_index("s") * NC + lax.axis_index("c")
        base = wid * BPW
        pltpu.sync_copy(tokens_hbm.at[pl.ds(base, BPW)], idx_v)

        def start_gather(i):
            b = i % NBUF
            return pltpu.async_copy(
                table_hbm.at[idx_v.at[pl.ds(i * CHUNK, CHUNK)]],
                bufs[b], gsems[b])

        def start_out(i):
            b = i % NBUF
            return pltpu.async_copy(
                bufs[b], out_hbm.at[pl.ds(base + i * CHUNK, CHUNK)],
                osems[b])

        gh = {i: start_gather(i) for i in range(min(NBUF, NCH))}
        oh = {}
        for i in range(NCH):
            gh[i].wait()
            oh[i] = start_out(i)
            if i + NBUF < NCH:
                oh[i].wait()
                gh[i + NBUF] = start_gather(i + NBUF)
        for i in range(max(0, NCH - NBUF), NCH):
            oh[i].wait()

    return pl.kernel(
        body,
        out_type=jax.ShapeDtypeStruct((N_TOKENS, D_MODEL), jnp.float32),
        mesh=mesh,
        scratch_types=[
            pltpu.VMEM((BPW,), jnp.int32),
            tuple(pltpu.VMEM((CHUNK, D_MODEL), jnp.float32)
                  for _ in range(NBUF)),
            tuple(pltpu.SemaphoreType.DMA for _ in range(NBUF)),
            tuple(pltpu.SemaphoreType.DMA for _ in range(NBUF)),
        ],
    )


def _make_tc_gather():
    def body(tok_smem, table_any, out_vmem, sem):
        g = pl.program_id(0)
        handles = []
        for r in range(ROWS_PER_STEP):
            idx = tok_smem[g * ROWS_PER_STEP + r]
            handles.append(
                pltpu.make_async_copy(
                    table_any.at[pl.ds(idx, 1), :],
                    out_vmem.at[pl.ds(r, 1), :],
                    sem,
                )
            )
            handles[-1].start()
        for h in handles:
            h.wait()

    grid_spec = pltpu.PrefetchScalarGridSpec(
        num_scalar_prefetch=1,
        grid=(TC_STEPS,),
        in_specs=[pl.BlockSpec(memory_space=pl.ANY)],
        out_specs=pl.BlockSpec((ROWS_PER_STEP, D_MODEL), lambda i, tok: (i, 0)),
        scratch_shapes=[pltpu.SemaphoreType.DMA],
    )

    def run(tokens_tc, W_E):
        return pl.pallas_call(
            body,
            grid_spec=grid_spec,
            out_shape=jax.ShapeDtypeStruct((N_TC, D_MODEL), jnp.float32),
        )(tokens_tc, W_E)

    return run


_sc_gather = _make_sc_gather()
_tc_gather = _make_tc_gather()


@jax.jit
def _gather_all(tokens_flat, W_E):
    from jax.experimental.compute_on import compute_on

    @compute_on('tpu_sparsecore')
    def sc_part(t, w):
        return _sc_gather(t, w)

    emb_full = sc_part(tokens_flat, W_E)
    emb_tc = _tc_gather(tokens_flat[N_SC:], W_E)
    return lax.dynamic_update_slice(emb_full, emb_tc, (N_SC, 0))


def kernel(tokens, W_E):
    B, S = tokens.shape
    tokens_flat = tokens.reshape(-1).astype(jnp.int32)
    emb = _gather_all(tokens_flat, W_E)
    return (tokens, emb.reshape(B, S, D_MODEL))


# speedup vs baseline: 1.4444x; 1.4381x over previous
import jax
import jax.numpy as jnp
from jax import lax
from jax.experimental import pallas as pl
from jax.experimental.pallas import tpu as pltpu
from jax.experimental.pallas import tpu_sc as plsc

D_MODEL = 768
BATCH = 4
SEQ = 4096
NC, NS = 2, 16
NW = NC * NS
BPW = BATCH * SEQ // NW
WPB = SEQ // BPW
CHUNK = 32
NCH = BPW // CHUNK
NBUF = 5


def _make_gather():
    mesh = plsc.VectorSubcoreMesh(core_axis_name="c", subcore_axis_name="s")

    @jax.jit
    def run(tokens, W_E):
        def body(tokens_hbm, table_hbm, out_hbm, idx_v, bufs, gsems, osems):
            wid = lax.axis_index("s") * NC + lax.axis_index("c")
            b = wid // WPB
            s0 = (wid % WPB) * BPW
            pltpu.sync_copy(tokens_hbm.at[b, pl.ds(s0, BPW)], idx_v)

            def start_gather(i):
                k = i % NBUF
                return pltpu.async_copy(
                    table_hbm.at[idx_v.at[pl.ds(i * CHUNK, CHUNK)]],
                    bufs[k], gsems[k])

            def start_out(i):
                k = i % NBUF
                return pltpu.async_copy(
                    bufs[k],
                    out_hbm.at[b, pl.ds(s0 + i * CHUNK, CHUNK)],
                    osems[k])

            gh = {i: start_gather(i) for i in range(min(NBUF, NCH))}
            oh = {}
            for i in range(NCH):
                gh[i].wait()
                oh[i] = start_out(i)
                if i + NBUF < NCH:
                    oh[i].wait()
                    gh[i + NBUF] = start_gather(i + NBUF)
            for i in range(max(0, NCH - NBUF), NCH):
                oh[i].wait()

        kfn = pl.kernel(
            body,
            out_type=jax.ShapeDtypeStruct((BATCH, SEQ, D_MODEL), jnp.float32),
            mesh=mesh,
            scratch_types=[
                pltpu.VMEM((BPW,), jnp.int32),
                tuple(pltpu.VMEM((CHUNK, D_MODEL), jnp.float32)
                      for _ in range(NBUF)),
                tuple(pltpu.SemaphoreType.DMA for _ in range(NBUF)),
                tuple(pltpu.SemaphoreType.DMA for _ in range(NBUF)),
            ],
        )
        return kfn(tokens, W_E)

    return run


_gather = _make_gather()


def kernel(tokens, W_E):
    emb = _gather(tokens.astype(jnp.int32), W_E)
    return (tokens, emb)


# speedup vs baseline: 1.4533x; 1.0062x over previous
import jax
import jax.numpy as jnp
from jax import lax
from jax.experimental import pallas as pl
from jax.experimental.pallas import tpu as pltpu
from jax.experimental.pallas import tpu_sc as plsc

D_MODEL = 768
BATCH = 4
SEQ = 4096
NC, NS = 2, 16
NW = NC * NS
BPW = BATCH * SEQ // NW
WPB = SEQ // BPW
CHUNK = 32
NCH = BPW // CHUNK
NBUF = 5


def _make_gather():
    mesh = plsc.VectorSubcoreMesh(core_axis_name="c", subcore_axis_name="s")

    @jax.jit
    def run(tokens, W_E):
        def body(tokens_hbm, table_hbm, out_hbm, tok_out_hbm,
                 idx_v, bufs, gsems, osems, tsem):
            wid = lax.axis_index("s") * NC + lax.axis_index("c")
            b = wid // WPB
            s0 = (wid % WPB) * BPW
            pltpu.sync_copy(tokens_hbm.at[b, pl.ds(s0, BPW)], idx_v)
            th = pltpu.async_copy(idx_v, tok_out_hbm.at[b, pl.ds(s0, BPW)],
                                  tsem)

            def start_gather(i):
                k = i % NBUF
                return pltpu.async_copy(
                    table_hbm.at[idx_v.at[pl.ds(i * CHUNK, CHUNK)]],
                    bufs[k], gsems[k])

            def start_out(i):
                k = i % NBUF
                return pltpu.async_copy(
                    bufs[k],
                    out_hbm.at[b, pl.ds(s0 + i * CHUNK, CHUNK)],
                    osems[k])

            gh = {i: start_gather(i) for i in range(min(NBUF, NCH))}
            oh = {}
            for i in range(NCH):
                gh[i].wait()
                oh[i] = start_out(i)
                if i + NBUF < NCH:
                    oh[i].wait()
                    gh[i + NBUF] = start_gather(i + NBUF)
            for i in range(max(0, NCH - NBUF), NCH):
                oh[i].wait()
            th.wait()

        kfn = pl.kernel(
            body,
            out_type=(
                jax.ShapeDtypeStruct((BATCH, SEQ, D_MODEL), jnp.float32),
                jax.ShapeDtypeStruct((BATCH, SEQ), jnp.int32),
            ),
            mesh=mesh,
            scratch_types=[
                pltpu.VMEM((BPW,), jnp.int32),
                tuple(pltpu.VMEM((CHUNK, D_MODEL), jnp.float32)
                      for _ in range(NBUF)),
                tuple(pltpu.SemaphoreType.DMA for _ in range(NBUF)),
                tuple(pltpu.SemaphoreType.DMA for _ in range(NBUF)),
                pltpu.SemaphoreType.DMA,
            ],
        )
        return kfn(tokens, W_E)

    return run


_gather = _make_gather()


def kernel(tokens, W_E):
    emb, tok_out = _gather(tokens.astype(jnp.int32), W_E)
    return (tok_out.astype(tokens.dtype), emb)


# speedup vs baseline: 1.4583x; 1.0034x over previous
import jax
import jax.numpy as jnp
from jax import lax
from jax.experimental import pallas as pl
from jax.experimental.pallas import tpu as pltpu
from jax.experimental.pallas import tpu_sc as plsc

D_MODEL = 768
BATCH = 4
SEQ = 4096
NC, NS = 2, 16
NW = NC * NS
BPW = BATCH * SEQ // NW
WPB = SEQ // BPW
CHUNK = 32
NCH = BPW // CHUNK
NBUF = 5


def _make_gather():
    mesh = plsc.VectorSubcoreMesh(core_axis_name="c", subcore_axis_name="s")

    @jax.jit
    def run(tokens, W_E):
        def body(tokens_hbm, table_hbm, out_hbm, tok_out_hbm,
                 idx_v, bufs, gsems, osems, tsem):
            wid = lax.axis_index("s") * NC + lax.axis_index("c")
            b = wid // WPB
            s0 = (wid % WPB) * BPW
            head = 256
            pltpu.sync_copy(tokens_hbm.at[b, pl.ds(s0, head)],
                            idx_v.at[pl.ds(0, head)])
            sh = pltpu.async_copy(tokens_hbm.at[b, pl.ds(s0 + head,
                                                         BPW - head)],
                                  idx_v.at[pl.ds(head, BPW - head)], tsem)

            def start_gather(i):
                k = i % NBUF
                return pltpu.async_copy(
                    table_hbm.at[idx_v.at[pl.ds(i * CHUNK, CHUNK)]],
                    bufs[k], gsems[k])

            def start_out(i):
                k = i % NBUF
                return pltpu.async_copy(
                    bufs[k],
                    out_hbm.at[b, pl.ds(s0 + i * CHUNK, CHUNK)],
                    osems[k])

            gh = {i: start_gather(i) for i in range(min(NBUF, NCH))}
            sh.wait()
            th = pltpu.async_copy(idx_v, tok_out_hbm.at[b, pl.ds(s0, BPW)],
                                  tsem)
            oh = {}
            for i in range(NCH):
                gh[i].wait()
                oh[i] = start_out(i)
                if i + NBUF < NCH:
                    oh[i].wait()
                    gh[i + NBUF] = start_gather(i + NBUF)
            for i in range(max(0, NCH - NBUF), NCH):
                oh[i].wait()
            th.wait()

        kfn = pl.kernel(
            body,
            out_type=(
                jax.ShapeDtypeStruct((BATCH, SEQ, D_MODEL), jnp.float32),
                jax.ShapeDtypeStruct((BATCH, SEQ), jnp.int32),
            ),
            mesh=mesh,
            scratch_types=[
                pltpu.VMEM((BPW,), jnp.int32),
                tuple(pltpu.VMEM((CHUNK, D_MODEL), jnp.float32)
                      for _ in range(NBUF)),
                tuple(pltpu.SemaphoreType.DMA for _ in range(NBUF)),
                tuple(pltpu.SemaphoreType.DMA for _ in range(NBUF)),
                pltpu.SemaphoreType.DMA,
            ],
        )
        return kfn(tokens, W_E)

    return run


_gather = _make_gather()


def kernel(tokens, W_E):
    emb, tok_out = _gather(tokens.astype(jnp.int32), W_E)
    return (tok_out.astype(tokens.dtype), emb)
